# baseline (device time: 36453 ns/iter reference)
import jax
import jax.numpy as jnp
from jax import lax
from jax.experimental import pallas as pl
from jax.experimental.pallas import tpu as pltpu

N_DEV = 4
CAP = 102


def kernel(x, router_W, route_idx, expert_W):
    n_tok, d_in = x.shape
    e_per, _, d_out = expert_W.shape

    def body(x_ref, ridx_ref, ew_ref, out_ref,
             wbuf, hbuf, wsend, wrecv, hsend, hrecv):
        my = lax.axis_index("i")

        bar = pltpu.get_barrier_semaphore()
        for off in (1, 2, 3):
            pl.semaphore_signal(
                bar, inc=1,
                device_id=((my + off) % N_DEV,),
                device_id_type=pl.DeviceIdType.MESH,
            )
        pl.semaphore_wait(bar, 3)

        x_bf = x_ref[...].astype(jnp.bfloat16)
        idx = ridx_ref[...][:, 0]

        wbuf[my] = ew_ref[...].astype(jnp.bfloat16)
        iota128 = lax.broadcasted_iota(jnp.int32, (n_tok, 128), 1)
        onehot128 = (idx[:, None] == iota128).astype(jnp.int32)
        hist = jnp.sum(onehot128, axis=0)
        hbuf[my] = hist[None, :]

        w_rdmas, h_rdmas = [], []
        for off in (1, 2, 3):
            tgt = (my + off) % N_DEV
            j = off - 1
            rh = pltpu.make_async_remote_copy(
                src_ref=hbuf.at[my], dst_ref=hbuf.at[my],
                send_sem=hsend.at[j], recv_sem=hrecv.at[j],
                device_id=(tgt,), device_id_type=pl.DeviceIdType.MESH,
            )
            rh.start()
            h_rdmas.append(rh)
            rw = pltpu.make_async_remote_copy(
                src_ref=wbuf.at[my], dst_ref=wbuf.at[my],
                send_sem=wsend.at[j], recv_sem=wrecv.at[j],
                device_id=(tgt,), device_id_type=pl.DeviceIdType.MESH,
            )
            rw.start()
            w_rdmas.append(rw)

        def packed_x(s):
            e_base = s * e_per
            iota4 = lax.broadcasted_iota(jnp.int32, (n_tok, e_per), 1)
            oh4 = (idx[:, None] == (e_base + iota4)).astype(jnp.bfloat16)
            return (x_bf[:, None, :] * oh4[:, :, None]).reshape(
                n_tok, e_per * d_in)

        def chunk_matmul(acc, xg, s):
            wg = wbuf[s].reshape(e_per * d_in, d_out)
            return acc + jnp.dot(xg, wg, preferred_element_type=jnp.float32)

        acc = chunk_matmul(
            jnp.zeros((n_tok, d_out), jnp.float32), packed_x(my), my)

        remote = [((my - off) % N_DEV, j) for off, j in
                  ((1, 0), (3, 2), (2, 1))]
        xgs = {j: packed_x(s) for s, j in remote}

        row_i = lax.broadcasted_iota(jnp.int32, (n_tok, n_tok), 0)
        col_i = lax.broadcasted_iota(jnp.int32, (n_tok, n_tok), 1)
        same = (idx[:, None] == idx[None, :]) & (col_i < row_i)
        rank = jnp.sum(same.astype(jnp.int32), axis=1)

        for j in range(3):
            s = (my - (j + 1)) % N_DEV
            rcv = pltpu.make_async_remote_copy(
                src_ref=hbuf.at[s], dst_ref=hbuf.at[s],
                send_sem=hsend.at[j], recv_sem=hrecv.at[j],
                device_id=(my,), device_id_type=pl.DeviceIdType.MESH,
            )
            rcv.wait_recv()

        allhist = hbuf[...][:, 0, :]
        shard_iota = lax.broadcasted_iota(jnp.int32, (N_DEV, 128), 0)
        prefix = jnp.sum(
            jnp.where(shard_iota < my, allhist, 0), axis=0)
        tok_prefix = jnp.sum(onehot128 * prefix[None, :], axis=1)
        keep = ((tok_prefix + rank) < CAP).astype(jnp.float32)

        for s, j in remote:
            rcv = pltpu.make_async_remote_copy(
                src_ref=wbuf.at[s], dst_ref=wbuf.at[s],
                send_sem=wsend.at[j], recv_sem=wrecv.at[j],
                device_id=(my,), device_id_type=pl.DeviceIdType.MESH,
            )
            rcv.wait_recv()
            acc = chunk_matmul(acc, xgs[j], s)

        out_ref[...] = acc * keep[:, None]

        for r in w_rdmas:
            r.wait_send()
        for r in h_rdmas:
            r.wait_send()

    return pl.pallas_call(
        body,
        out_shape=jax.ShapeDtypeStruct((n_tok, d_out), jnp.float32),
        in_specs=[
            pl.BlockSpec(memory_space=pltpu.VMEM),
            pl.BlockSpec(memory_space=pltpu.VMEM),
            pl.BlockSpec(memory_space=pltpu.VMEM),
        ],
        out_specs=pl.BlockSpec(memory_space=pltpu.VMEM),
        scratch_shapes=[
            pltpu.VMEM((N_DEV, e_per, d_in, d_out), jnp.bfloat16),
            pltpu.VMEM((N_DEV, 1, 128), jnp.int32),
            pltpu.SemaphoreType.DMA((3,)),
            pltpu.SemaphoreType.DMA((3,)),
            pltpu.SemaphoreType.DMA((3,)),
            pltpu.SemaphoreType.DMA((3,)),
        ],
        compiler_params=pltpu.CompilerParams(collective_id=0),
    )(x, route_idx, expert_W)


# device time: 33765 ns/iter; 1.0796x vs baseline; 1.0796x over previous
import jax
import jax.numpy as jnp
from jax import lax
from jax.experimental import pallas as pl
from jax.experimental.pallas import tpu as pltpu

N_DEV = 4
CAP = 102


def kernel(x, router_W, route_idx, expert_W):
    n_tok, d_in = x.shape
    e_per, _, d_out = expert_W.shape

    def body(x_ref, ridx_ref, ew_ref, out_ref,
             wbuf, hbuf, wsend, wrecv, hsend, hrecv):
        my = lax.axis_index("i")

        bar = pltpu.get_barrier_semaphore()
        for off in (1, 2, 3):
            pl.semaphore_signal(
                bar, inc=1,
                device_id=((my + off) % N_DEV,),
                device_id_type=pl.DeviceIdType.MESH,
            )
        pl.semaphore_wait(bar, 3)

        x_bf = x_ref[...].astype(jnp.bfloat16)
        idx = ridx_ref[...][:, 0]

        iota128 = lax.broadcasted_iota(jnp.int32, (n_tok, 128), 1)
        onehot128 = (idx[:, None] == iota128).astype(jnp.int32)
        hist = jnp.sum(onehot128, axis=0)
        hbuf[my] = hist[None, :]
        h_rdmas = []
        for off in (1, 2, 3):
            tgt = (my + off) % N_DEV
            j = off - 1
            rh = pltpu.make_async_remote_copy(
                src_ref=hbuf.at[my], dst_ref=hbuf.at[my],
                send_sem=hsend.at[j], recv_sem=hrecv.at[j],
                device_id=(tgt,), device_id_type=pl.DeviceIdType.MESH,
            )
            rh.start()
            h_rdmas.append(rh)

        w_rdmas = []
        half = e_per // 2
        for h in (0, 1):
            wbuf[my, h * half:(h + 1) * half] = (
                ew_ref[h * half:(h + 1) * half].astype(jnp.bfloat16))
            for off in (1, 2, 3):
                tgt = (my + off) % N_DEV
                j = off - 1
                rw = pltpu.make_async_remote_copy(
                    src_ref=wbuf.at[my, h * half:(h + 1) * half],
                    dst_ref=wbuf.at[my, h * half:(h + 1) * half],
                    send_sem=wsend.at[j, h], recv_sem=wrecv.at[j, h],
                    device_id=(tgt,), device_id_type=pl.DeviceIdType.MESH,
                )
                rw.start()
                w_rdmas.append(rw)

        def packed_x(s, h):
            e_base = s * e_per + h * half
            iota_h = lax.broadcasted_iota(jnp.int32, (n_tok, half), 1)
            oh = (idx[:, None] == (e_base + iota_h)).astype(jnp.bfloat16)
            return (x_bf[:, None, :] * oh[:, :, None]).reshape(
                n_tok, half * d_in)

        def half_matmul(acc, xg, s, h):
            wg = wbuf[s, h * half:(h + 1) * half].reshape(
                half * d_in, d_out)
            return acc + jnp.dot(xg, wg, preferred_element_type=jnp.float32)

        acc = jnp.zeros((n_tok, d_out), jnp.float32)
        for h in (0, 1):
            acc = half_matmul(acc, packed_x(my, h), my, h)

        remote = [((my - off) % N_DEV, j) for off, j in
                  ((1, 0), (3, 2), (2, 1))]
        xgs = {(j, h): packed_x(s, h) for s, j in remote for h in (0, 1)}

        row_i = lax.broadcasted_iota(jnp.int32, (n_tok, n_tok), 0)
        col_i = lax.broadcasted_iota(jnp.int32, (n_tok, n_tok), 1)
        same = (idx[:, None] == idx[None, :]) & (col_i < row_i)
        rank = jnp.sum(same.astype(jnp.int32), axis=1)

        for j in range(3):
            s = (my - (j + 1)) % N_DEV
            rcv = pltpu.make_async_remote_copy(
                src_ref=hbuf.at[s], dst_ref=hbuf.at[s],
                send_sem=hsend.at[j], recv_sem=hrecv.at[j],
                device_id=(my,), device_id_type=pl.DeviceIdType.MESH,
            )
            rcv.wait_recv()

        allhist = hbuf[...][:, 0, :]
        shard_iota = lax.broadcasted_iota(jnp.int32, (N_DEV, 128), 0)
        prefix = jnp.sum(
            jnp.where(shard_iota < my, allhist, 0), axis=0)
        tok_prefix = jnp.sum(onehot128 * prefix[None, :], axis=1)
        keep = ((tok_prefix + rank) < CAP).astype(jnp.float32)

        for h in (0, 1):
            for s, j in remote:
                rcv = pltpu.make_async_remote_copy(
                    src_ref=wbuf.at[s, h * half:(h + 1) * half],
                    dst_ref=wbuf.at[s, h * half:(h + 1) * half],
                    send_sem=wsend.at[j, h], recv_sem=wrecv.at[j, h],
                    device_id=(my,), device_id_type=pl.DeviceIdType.MESH,
                )
                rcv.wait_recv()
                acc = half_matmul(acc, xgs[(j, h)], s, h)

        out_ref[...] = acc * keep[:, None]

        for r in w_rdmas:
            r.wait_send()
        for r in h_rdmas:
            r.wait_send()

    return pl.pallas_call(
        body,
        out_shape=jax.ShapeDtypeStruct((n_tok, d_out), jnp.float32),
        in_specs=[
            pl.BlockSpec(memory_space=pltpu.VMEM),
            pl.BlockSpec(memory_space=pltpu.VMEM),
            pl.BlockSpec(memory_space=pltpu.VMEM),
        ],
        out_specs=pl.BlockSpec(memory_space=pltpu.VMEM),
        scratch_shapes=[
            pltpu.VMEM((N_DEV, e_per, d_in, d_out), jnp.bfloat16),
            pltpu.VMEM((N_DEV, 1, 128), jnp.int32),
            pltpu.SemaphoreType.DMA((3, 2)),
            pltpu.SemaphoreType.DMA((3, 2)),
            pltpu.SemaphoreType.DMA((3,)),
            pltpu.SemaphoreType.DMA((3,)),
        ],
        compiler_params=pltpu.CompilerParams(collective_id=0),
    )(x, route_idx, expert_W)


# device time: 21958 ns/iter; 1.6601x vs baseline; 1.5377x over previous
import jax
import jax.numpy as jnp
from jax import lax
from jax.experimental import pallas as pl
from jax.experimental.pallas import tpu as pltpu

N_DEV = 4
CAP = 102


def kernel(x, router_W, route_idx, expert_W):
    n_tok, d_in = x.shape
    e_per, _, d_out = expert_W.shape

    def body(x_ref, ridx_ref, ew_ref, out_ref,
             wbuf, hbuf, wsend, wrecv, hsend, hrecv):
        my = lax.axis_index("i")

        bar = pltpu.get_barrier_semaphore()
        for off in (1, 2, 3):
            pl.semaphore_signal(
                bar, inc=1,
                device_id=((my + off) % N_DEV,),
                device_id_type=pl.DeviceIdType.MESH,
            )
        pl.semaphore_wait(bar, 3)

        x_bf = x_ref[...].astype(jnp.bfloat16)
        idx = ridx_ref[...][:, 0]

        scal = jnp.maximum(
            jnp.max(jnp.abs(ew_ref[...]), axis=(1, 2)), 1e-20) / 127.0
        inv_scal = 1.0 / scal

        iota128 = lax.broadcasted_iota(jnp.int32, (n_tok, 128), 1)
        onehot128 = (idx[:, None] == iota128).astype(jnp.float32)
        hist = jnp.sum(onehot128, axis=0)
        lane4 = lax.broadcasted_iota(jnp.int32, (e_per, 128), 1)
        exp4 = lax.broadcasted_iota(jnp.int32, (e_per, 128), 0)
        scal128 = jnp.sum(
            jnp.where(lane4 == 16 + e_per * my + exp4, scal[:, None], 0.0),
            axis=0)
        hbuf[my] = (hist + scal128)[None, :]
        h_rdmas = []
        for off in (1, 2, 3):
            tgt = (my + off) % N_DEV
            j = off - 1
            rh = pltpu.make_async_remote_copy(
                src_ref=hbuf.at[my], dst_ref=hbuf.at[my],
                send_sem=hsend.at[j], recv_sem=hrecv.at[j],
                device_id=(tgt,), device_id_type=pl.DeviceIdType.MESH,
            )
            rh.start()
            h_rdmas.append(rh)

        w_rdmas = []
        half = e_per // 2
        for h in (0, 1):
            wbuf[my, h * half:(h + 1) * half] = lax.round(
                ew_ref[h * half:(h + 1) * half]
                * inv_scal[h * half:(h + 1) * half, None, None],
                lax.RoundingMethod.TO_NEAREST_EVEN,
            ).astype(jnp.int8)
            for off in (1, 2, 3):
                tgt = (my + off) % N_DEV
                j = off - 1
                rw = pltpu.make_async_remote_copy(
                    src_ref=wbuf.at[my, h * half:(h + 1) * half],
                    dst_ref=wbuf.at[my, h * half:(h + 1) * half],
                    send_sem=wsend.at[j, h], recv_sem=wrecv.at[j, h],
                    device_id=(tgt,), device_id_type=pl.DeviceIdType.MESH,
                )
                rw.start()
                w_rdmas.append(rw)

        def packed_x(s, h):
            e_base = s * e_per + h * half
            iota_h = lax.broadcasted_iota(jnp.int32, (n_tok, half), 1)
            oh = (idx[:, None] == (e_base + iota_h)).astype(jnp.bfloat16)
            return (x_bf[:, None, :] * oh[:, :, None]).reshape(
                n_tok, half * d_in)

        def half_matmul(acc, xg, s, h):
            wg = wbuf[s, h * half:(h + 1) * half].astype(
                jnp.bfloat16).reshape(half * d_in, d_out)
            return acc + jnp.dot(xg, wg, preferred_element_type=jnp.float32)

        acc = jnp.zeros((n_tok, d_out), jnp.float32)
        for h in (0, 1):
            acc = half_matmul(acc, packed_x(my, h), my, h)

        remote = [((my - off) % N_DEV, j) for off, j in
                  ((1, 0), (3, 2), (2, 1))]
        xgs = {(j, h): packed_x(s, h) for s, j in remote for h in (0, 1)}

        row_i = lax.broadcasted_iota(jnp.int32, (n_tok, n_tok), 0)
        col_i = lax.broadcasted_iota(jnp.int32, (n_tok, n_tok), 1)
        same = (idx[:, None] == idx[None, :]) & (col_i < row_i)
        rank = jnp.sum(same.astype(jnp.int32), axis=1)

        for j in range(3):
            s = (my - (j + 1)) % N_DEV
            rcv = pltpu.make_async_remote_copy(
                src_ref=hbuf.at[s], dst_ref=hbuf.at[s],
                send_sem=hsend.at[j], recv_sem=hrecv.at[j],
                device_id=(my,), device_id_type=pl.DeviceIdType.MESH,
            )
            rcv.wait_recv()

        allhist = hbuf[...][:, 0, :]
        shard_iota = lax.broadcasted_iota(jnp.int32, (N_DEV, 128), 0)
        prefix = jnp.sum(
            jnp.where(shard_iota < my, allhist, 0.0), axis=0)
        tok_prefix = jnp.sum(onehot128 * prefix[None, :], axis=1)
        total = jnp.sum(allhist, axis=0)
        oh_scale = (iota128 == (idx[:, None] + 16)).astype(jnp.float32)
        tok_scale = jnp.sum(oh_scale * total[None, :], axis=1)
        keep = jnp.where(
            (tok_prefix + rank.astype(jnp.float32)) < CAP, tok_scale, 0.0)

        for h in (0, 1):
            for s, j in remote:
                rcv = pltpu.make_async_remote_copy(
                    src_ref=wbuf.at[s, h * half:(h + 1) * half],
                    dst_ref=wbuf.at[s, h * half:(h + 1) * half],
                    send_sem=wsend.at[j, h], recv_sem=wrecv.at[j, h],
                    device_id=(my,), device_id_type=pl.DeviceIdType.MESH,
                )
                rcv.wait_recv()
                acc = half_matmul(acc, xgs[(j, h)], s, h)

        out_ref[...] = acc * keep[:, None]

        for r in w_rdmas:
            r.wait_send()
        for r in h_rdmas:
            r.wait_send()

    return pl.pallas_call(
        body,
        out_shape=jax.ShapeDtypeStruct((n_tok, d_out), jnp.float32),
        in_specs=[
            pl.BlockSpec(memory_space=pltpu.VMEM),
            pl.BlockSpec(memory_space=pltpu.VMEM),
            pl.BlockSpec(memory_space=pltpu.VMEM),
        ],
        out_specs=pl.BlockSpec(memory_space=pltpu.VMEM),
        scratch_shapes=[
            pltpu.VMEM((N_DEV, e_per, d_in, d_out), jnp.int8),
            pltpu.VMEM((N_DEV, 1, 128), jnp.float32),
            pltpu.SemaphoreType.DMA((3, 2)),
            pltpu.SemaphoreType.DMA((3, 2)),
            pltpu.SemaphoreType.DMA((3,)),
            pltpu.SemaphoreType.DMA((3,)),
        ],
        compiler_params=pltpu.CompilerParams(collective_id=0),
    )(x, route_idx, expert_W)
